# baseline (speedup 1.0000x reference)
import jax
import jax.numpy as jnp
from jax import lax
from jax.experimental import pallas as pl
from jax.experimental.pallas import tpu as pltpu

ROWS = 128
COLS = 100000
LANE = 128
BLK_A = 4096
BLK_W = 4096


def _threefry_bits(p):
    ks0 = jnp.uint32(0)
    ks1 = jnp.uint32(42)
    ks2 = ks0 ^ ks1 ^ jnp.uint32(0x1BD11BDA)
    ks = (ks0, ks1, ks2)
    rots = ((13, 15, 26, 6), (17, 29, 16, 24))

    def rotl(v, r):
        return (v << jnp.uint32(r)) | (v >> jnp.uint32(32 - r))

    x1i = p + ks1
    x0 = x1i
    x1 = x0 ^ rotl(x1i, rots[0][0])
    for r in rots[0][1:]:
        x0 = x0 + x1
        x1 = x0 ^ rotl(x1, r)
    x0 = x0 + ks[1]
    x1 = x1 + ks[2] + jnp.uint32(1)
    for i in range(1, 5):
        for r in rots[i % 2]:
            x0 = x0 + x1
            x1 = x0 ^ rotl(x1, r)
        x0 = x0 + ks[(i + 1) % 3]
        x1 = x1 + ks[(i + 2) % 3] + jnp.uint32(i + 1)
    return x0 ^ x1


def _gumbel(p):
    bits = _threefry_bits(p)
    fb = (bits >> jnp.uint32(9)) | jnp.uint32(0x3F800000)
    f = lax.bitcast_convert_type(fb, jnp.float32) - jnp.float32(1.0)
    u = jnp.maximum(f, jnp.float32(1e-20))
    return -jnp.log(-jnp.log(u))


def _argmax_kernel(x_ref, idx_ref, val_s, col_s):
    j = pl.program_id(0)
    nb = pl.num_programs(0)

    @pl.when(j == 0)
    def _():
        val_s[...] = jnp.full((ROWS, LANE), -jnp.inf, jnp.float32)
        col_s[...] = jnp.zeros((ROWS, LANE), jnp.int32)

    v = x_ref[...]
    rows = lax.broadcasted_iota(jnp.uint32, v.shape, 0)
    cols_i = lax.broadcasted_iota(jnp.int32, v.shape, 1) + j * BLK_A
    p = rows * jnp.uint32(COLS) + cols_i.astype(jnp.uint32)
    val = v + _gumbel(p)
    val = jnp.where(cols_i < COLS, val, -jnp.inf)

    acc_v = val_s[...]
    acc_c = col_s[...]
    for k in range(BLK_A // LANE):
        chunk = val[:, k * LANE:(k + 1) * LANE]
        ccol = cols_i[:, k * LANE:(k + 1) * LANE]
        better = chunk > acc_v
        acc_v = jnp.where(better, chunk, acc_v)
        acc_c = jnp.where(better, ccol, acc_c)
    val_s[...] = acc_v
    col_s[...] = acc_c

    @pl.when(j == nb - 1)
    def _():
        m = jnp.max(acc_v, axis=1, keepdims=True)
        cand = jnp.where(acc_v == m, acc_c, jnp.int32(2**31 - 1))
        idx_ref[...] = jnp.min(cand, axis=1, keepdims=True)


def _onehot_kernel(idx_ref, o_ref):
    j = pl.program_id(0)
    cols = lax.broadcasted_iota(jnp.int32, o_ref.shape, 1) + j * BLK_W
    o_ref[...] = (cols == idx_ref[...]).astype(jnp.float32)


def kernel(x):
    idx = pl.pallas_call(
        _argmax_kernel,
        grid=(pl.cdiv(COLS, BLK_A),),
        in_specs=[pl.BlockSpec((ROWS, BLK_A), lambda j: (0, j))],
        out_specs=pl.BlockSpec((ROWS, 1), lambda j: (0, 0)),
        out_shape=jax.ShapeDtypeStruct((ROWS, 1), jnp.int32),
        scratch_shapes=[
            pltpu.VMEM((ROWS, LANE), jnp.float32),
            pltpu.VMEM((ROWS, LANE), jnp.int32),
        ],
    )(x)
    out = pl.pallas_call(
        _onehot_kernel,
        grid=(pl.cdiv(COLS, BLK_W),),
        in_specs=[pl.BlockSpec((ROWS, 1), lambda j: (0, 0))],
        out_specs=pl.BlockSpec((ROWS, BLK_W), lambda j: (0, j)),
        out_shape=jax.ShapeDtypeStruct((ROWS, COLS), jnp.float32),
    )(idx)
    return out


# speedup vs baseline: 1.5201x; 1.5201x over previous
import jax
import jax.numpy as jnp
from jax import lax
from jax.experimental import pallas as pl
from jax.experimental.pallas import tpu as pltpu

ROWS = 128
COLS = 100000
SUB = 64
BLK_A = 2048
BLK_W = 4096


def _threefry_bits(p):
    ks0 = jnp.uint32(0)
    ks1 = jnp.uint32(42)
    ks2 = ks0 ^ ks1 ^ jnp.uint32(0x1BD11BDA)
    ks = (ks0, ks1, ks2)
    rots = ((13, 15, 26, 6), (17, 29, 16, 24))

    def rotl(v, r):
        return (v << jnp.uint32(r)) | (v >> jnp.uint32(32 - r))

    x1i = p + ks1
    x0 = x1i
    x1 = x0 ^ rotl(x1i, rots[0][0])
    for r in rots[0][1:]:
        x0 = x0 + x1
        x1 = x0 ^ rotl(x1, r)
    x0 = x0 + ks[1]
    x1 = x1 + ks[2] + jnp.uint32(1)
    for i in range(1, 5):
        for r in rots[i % 2]:
            x0 = x0 + x1
            x1 = x0 ^ rotl(x1, r)
        x0 = x0 + ks[(i + 1) % 3]
        x1 = x1 + ks[(i + 2) % 3] + jnp.uint32(i + 1)
    return x0 ^ x1


def _gumbel(p):
    bits = _threefry_bits(p)
    fb = (bits >> jnp.uint32(9)) | jnp.uint32(0x3F800000)
    f = lax.bitcast_convert_type(fb, jnp.float32) - jnp.float32(1.0)
    u = jnp.maximum(f, jnp.float32(1e-20))
    return -jnp.log(-jnp.log(u))


def _argmax_kernel(x_ref, idx_ref, val_s, col_s):
    j = pl.program_id(0)
    nb = pl.num_programs(0)

    @pl.when(j == 0)
    def _():
        val_s[...] = jnp.full((SUB, ROWS), -jnp.inf, jnp.float32)
        col_s[...] = jnp.zeros((SUB, ROWS), jnp.int32)

    lanes_r = lax.broadcasted_iota(jnp.uint32, (SUB, ROWS), 1)
    subi = lax.broadcasted_iota(jnp.int32, (SUB, ROWS), 0)
    rbase = lanes_r * jnp.uint32(COLS)

    acc_v = val_s[...]
    acc_c = col_s[...]
    for k in range(BLK_A // SUB):
        xs = x_ref[k * SUB:(k + 1) * SUB, :]
        ccol = subi + (j * BLK_A + k * SUB)
        p = rbase + ccol.astype(jnp.uint32)
        chunk = xs + _gumbel(p)
        chunk = jnp.where(ccol < COLS, chunk, -jnp.inf)
        better = chunk > acc_v
        acc_v = jnp.where(better, chunk, acc_v)
        acc_c = jnp.where(better, ccol, acc_c)
    val_s[...] = acc_v
    col_s[...] = acc_c

    @pl.when(j == nb - 1)
    def _():
        m = jnp.max(acc_v, axis=0, keepdims=True)
        cand = jnp.where(acc_v == m, acc_c, jnp.int32(2**31 - 1))
        idx_ref[...] = jnp.min(cand, axis=0, keepdims=True)


def _onehot_kernel(idx_ref, o_ref):
    j = pl.program_id(0)
    rowid = lax.broadcasted_iota(jnp.int32, (BLK_W, ROWS), 0) + j * BLK_W
    o_ref[...] = (rowid == idx_ref[...]).astype(jnp.float32)


def kernel(x):
    xt = x.T
    idx = pl.pallas_call(
        _argmax_kernel,
        grid=(pl.cdiv(COLS, BLK_A),),
        in_specs=[pl.BlockSpec((BLK_A, ROWS), lambda j: (j, 0))],
        out_specs=pl.BlockSpec((1, ROWS), lambda j: (0, 0)),
        out_shape=jax.ShapeDtypeStruct((1, ROWS), jnp.int32),
        scratch_shapes=[
            pltpu.VMEM((SUB, ROWS), jnp.float32),
            pltpu.VMEM((SUB, ROWS), jnp.int32),
        ],
    )(xt)
    out_t = pl.pallas_call(
        _onehot_kernel,
        grid=(pl.cdiv(COLS, BLK_W),),
        in_specs=[pl.BlockSpec((1, ROWS), lambda j: (0, 0))],
        out_specs=pl.BlockSpec((BLK_W, ROWS), lambda j: (j, 0)),
        out_shape=jax.ShapeDtypeStruct((COLS, ROWS), jnp.float32),
    )(idx)
    return out_t.T
